# Initial kernel scaffold; baseline (speedup 1.0000x reference)
import jax
import jax.numpy as jnp
from jax.experimental import pallas as pl


def kernel(edge_type, table):
    raise NotImplementedError("write your pallas kernel here")


# speedup vs baseline: 3.7726x; 3.7726x over previous
import functools

import jax
import jax.numpy as jnp
from jax import lax
from jax.experimental import pallas as pl
from jax.experimental.pallas import tpu as pltpu
from jax.experimental.pallas import tpu_sc as plsc

N_EDGES = 1600000
EMBED = 32
NUM_CORES = 2
NUM_SUBCORES = 16
NUM_WORKERS = NUM_CORES * NUM_SUBCORES
PER_WORKER = N_EDGES // NUM_WORKERS
CHUNK = 1000
N_CHUNKS = PER_WORKER // CHUNK
NBUF = 2
OUTER = N_CHUNKS // NBUF


def _body(idx_hbm, table_hbm, out_hbm, idx_v, rows_v, sem_i, sem_g, sem_o):
    wid = lax.axis_index("s") * NUM_CORES + lax.axis_index("c")
    base = wid * PER_WORKER

    def idx_cp(g, b):
        return pltpu.make_async_copy(
            idx_hbm.at[pl.ds(base + g * CHUNK, CHUNK)], idx_v.at[b], sem_i.at[b])

    def gath(b):
        return pltpu.make_async_copy(table_hbm.at[idx_v.at[b]], rows_v.at[b],
                                     sem_g.at[b])

    def out_cp(g, b):
        return pltpu.make_async_copy(
            rows_v.at[b], out_hbm.at[pl.ds(base + g * CHUNK, CHUNK)], sem_o.at[b])

    for b in range(NBUF):
        idx_cp(b, b).start()

    @pl.loop(0, OUTER)
    def step(o):
        for b in range(NBUF):
            g = o * NBUF + b

            @pl.when(o >= 1)
            def _():
                out_cp(g - NBUF, b).wait()

            idx_cp(g, b).wait()
            gath(b).start()
            gath(b).wait()

            @pl.when(o + 1 < OUTER)
            def _():
                idx_cp(g + NBUF, b).start()

            out_cp(g, b).start()

    for b in range(NBUF):
        out_cp(N_CHUNKS - NBUF + b, b).wait()


@functools.partial(jax.jit, static_argnames=())
def kernel(edge_type, table):
    idx = edge_type.astype(jnp.int32)
    mesh = plsc.VectorSubcoreMesh(
        core_axis_name="c", subcore_axis_name="s", num_cores=NUM_CORES
    )
    run = pl.kernel(
        _body,
        out_type=jax.ShapeDtypeStruct((N_EDGES, EMBED), jnp.float32),
        mesh=mesh,
        scratch_types=[
            pltpu.VMEM((NBUF, CHUNK), jnp.int32),
            pltpu.VMEM((NBUF, CHUNK, EMBED), jnp.float32),
            pltpu.SemaphoreType.DMA((NBUF,)),
            pltpu.SemaphoreType.DMA((NBUF,)),
            pltpu.SemaphoreType.DMA((NBUF,)),
        ],
        compiler_params=pltpu.CompilerParams(use_tc_tiling_on_sc=False),
    )
    return run(idx, table)


# speedup vs baseline: 4.4822x; 1.1881x over previous
import functools

import jax
import jax.numpy as jnp
from jax import lax
from jax.experimental import pallas as pl
from jax.experimental.pallas import tpu as pltpu
from jax.experimental.pallas import tpu_sc as plsc

N_EDGES = 1600000
EMBED = 32
NUM_ROWS = 1000
NUM_CORES = 2
NUM_SUBCORES = 16
NUM_WORKERS = NUM_CORES * NUM_SUBCORES
PER_WORKER = N_EDGES // NUM_WORKERS
CHUNK = 1000
N_CHUNKS = PER_WORKER // CHUNK
NBUF = 2
OUTER = N_CHUNKS // NBUF


def _body(idx_hbm, table_hbm, out_hbm, idx_v, rows_v, sem_i, sem_g, sem_o):
    wid = lax.axis_index("s") * NUM_CORES + lax.axis_index("c")
    base = wid * PER_WORKER
    tbl_w = table_hbm.at[pl.ds(wid * NUM_ROWS, NUM_ROWS)]

    def idx_cp(g, b):
        return pltpu.make_async_copy(
            idx_hbm.at[pl.ds(base + g * CHUNK, CHUNK)], idx_v.at[b], sem_i.at[b])

    def gath(b):
        return pltpu.make_async_copy(tbl_w.at[idx_v.at[b]], rows_v.at[b],
                                     sem_g.at[b])

    def out_cp(g, b):
        return pltpu.make_async_copy(
            rows_v.at[b], out_hbm.at[pl.ds(base + g * CHUNK, CHUNK)], sem_o.at[b])

    for b in range(NBUF):
        idx_cp(b, b).start()

    @pl.loop(0, OUTER)
    def step(o):
        for b in range(NBUF):
            g = o * NBUF + b

            @pl.when(o >= 1)
            def _():
                out_cp(g - NBUF, b).wait()

            idx_cp(g, b).wait()
            gath(b).start()
            gath(b).wait()

            @pl.when(o + 1 < OUTER)
            def _():
                idx_cp(g + NBUF, b).start()

            out_cp(g, b).start()

    for b in range(NBUF):
        out_cp(N_CHUNKS - NBUF + b, b).wait()


@functools.partial(jax.jit, static_argnames=())
def kernel(edge_type, table):
    idx = edge_type.astype(jnp.int32)
    table_rep = jnp.tile(table, (NUM_WORKERS, 1))
    mesh = plsc.VectorSubcoreMesh(
        core_axis_name="c", subcore_axis_name="s", num_cores=NUM_CORES
    )
    run = pl.kernel(
        _body,
        out_type=jax.ShapeDtypeStruct((N_EDGES, EMBED), jnp.float32),
        mesh=mesh,
        scratch_types=[
            pltpu.VMEM((NBUF, CHUNK), jnp.int32),
            pltpu.VMEM((NBUF, CHUNK, EMBED), jnp.float32),
            pltpu.SemaphoreType.DMA((NBUF,)),
            pltpu.SemaphoreType.DMA((NBUF,)),
            pltpu.SemaphoreType.DMA((NBUF,)),
        ],
        compiler_params=pltpu.CompilerParams(use_tc_tiling_on_sc=False),
    )
    return run(idx, table_rep)


# speedup vs baseline: 4.5190x; 1.0082x over previous
import functools

import jax
import jax.numpy as jnp
from jax import lax
from jax.experimental import pallas as pl
from jax.experimental.pallas import tpu as pltpu
from jax.experimental.pallas import tpu_sc as plsc

N_EDGES = 1600000
EMBED = 32
NUM_ROWS = 1000
NUM_CORES = 2
NUM_SUBCORES = 16
NUM_WORKERS = NUM_CORES * NUM_SUBCORES
PER_WORKER = N_EDGES // NUM_WORKERS
CHUNK = 400
N_CHUNKS = PER_WORKER // CHUNK
NBUF = 4
LAG = 2


def _body(idx_hbm, table_hbm, out_hbm, idx_v, rows_v, sem_i, sem_g, sem_o):
    wid = lax.axis_index("s") * NUM_CORES + lax.axis_index("c")
    base = wid * PER_WORKER
    tbl_w = table_hbm.at[pl.ds(wid * NUM_ROWS, NUM_ROWS)]

    def idx_cp(g, b):
        return pltpu.make_async_copy(
            idx_hbm.at[pl.ds(base + g * CHUNK, CHUNK)], idx_v.at[b], sem_i.at[b])

    def gath(b):
        return pltpu.make_async_copy(tbl_w.at[idx_v.at[b]], rows_v.at[b],
                                     sem_g.at[b])

    def out_cp(g, b):
        return pltpu.make_async_copy(
            rows_v.at[b], out_hbm.at[pl.ds(base + g * CHUNK, CHUNK)], sem_o.at[b])

    for b in range(min(NBUF, N_CHUNKS)):
        idx_cp(b, b).start()

    @pl.loop(0, N_CHUNKS + LAG)
    def step(g):
        b = lax.rem(g, NBUF)

        @pl.when(g < N_CHUNKS)
        def _():
            @pl.when(g >= NBUF)
            def _():
                out_cp(g - NBUF, b).wait()

            idx_cp(g, b).wait()
            gath(b).start()

        @pl.when(g >= LAG)
        def _():
            gl = g - LAG
            bl = lax.rem(gl, NBUF)
            gath(bl).wait()

            @pl.when(gl + NBUF < N_CHUNKS)
            def _():
                idx_cp(gl + NBUF, bl).start()

            out_cp(gl, bl).start()

    for k in range(NBUF):
        g = N_CHUNKS - NBUF + k
        out_cp(g, g % NBUF).wait()


@functools.partial(jax.jit, static_argnames=())
def kernel(edge_type, table):
    idx = edge_type.astype(jnp.int32)
    table_rep = jnp.tile(table, (NUM_WORKERS, 1))
    mesh = plsc.VectorSubcoreMesh(
        core_axis_name="c", subcore_axis_name="s", num_cores=NUM_CORES
    )
    run = pl.kernel(
        _body,
        out_type=jax.ShapeDtypeStruct((N_EDGES, EMBED), jnp.float32),
        mesh=mesh,
        scratch_types=[
            pltpu.VMEM((NBUF, CHUNK), jnp.int32),
            pltpu.VMEM((NBUF, CHUNK, EMBED), jnp.float32),
            pltpu.SemaphoreType.DMA((NBUF,)),
            pltpu.SemaphoreType.DMA((NBUF,)),
            pltpu.SemaphoreType.DMA((NBUF,)),
        ],
        compiler_params=pltpu.CompilerParams(use_tc_tiling_on_sc=False),
    )
    return run(idx, table_rep)
